# baseline (speedup 1.0000x reference)
import jax
import jax.numpy as jnp
from jax.experimental import pallas as pl

N = 100000
C = 80
LOSS_WEIGHT = 1.0

_LOG1P = (
    0.000925183135894625,
    0.9797525353876084,
    -0.39353455735239506,
    0.10668430401703402,
)

BLK = 4000
GRID = N // BLK
SUB = 160
NSUB = BLK // SUB


def _mxu_col(row):
    return jax.lax.dot_general(
        row, jnp.ones((1, 1), jnp.float32),
        ((( 0,), (0,)), ((), ())),
        preferred_element_type=jnp.float32)


def _body(x_ref, lab_ref, sc_ref, out_ref):
    cols = jax.lax.broadcasted_iota(jnp.int32, (SUB, C), 1).astype(jnp.float32)

    lab_col = _mxu_col(lab_ref[0, :, :].astype(jnp.float32))
    sc_col = _mxu_col(sc_ref[0, :, :])

    acc = jnp.zeros((SUB, C), jnp.float32)
    for j in range(NSUB):
        x = x_ref[pl.ds(j * SUB, SUB), :]
        lab = jax.lax.slice(lab_col, (j * SUB, 0), (j * SUB + SUB, 1))
        sc = jax.lax.slice(sc_col, (j * SUB, 0), (j * SUB + SUB, 1))

        u = jnp.exp(-jnp.abs(x))
        p = jnp.float32(_LOG1P[-1])
        for c in _LOG1P[-2::-1]:
            p = p * u + jnp.float32(c)
        sp = jnp.maximum(x, 0.0) + p
        t = 1.0 / (1.0 + u)
        s = jnp.where(x >= 0, t, u * t)
        negv = (0.75 * sp) * (s * s)

        w = sc - x
        pos_loss = (sp - x * sc) * (w * w)
        onehot = cols == lab
        acc = acc + negv + jnp.where(onehot, pos_loss - negv, 0.0)

    out_ref[...] = jnp.sum(acc).reshape(1, 1, 1)


def kernel(pred, label, score):
    parts = pl.pallas_call(
        _body,
        grid=(GRID,),
        in_specs=[
            pl.BlockSpec((BLK, C), lambda i: (i, 0)),
            pl.BlockSpec((1, 1, BLK), lambda i: (i, 0, 0)),
            pl.BlockSpec((1, 1, BLK), lambda i: (i, 0, 0)),
        ],
        out_specs=pl.BlockSpec((1, 1, 1), lambda i: (i, 0, 0)),
        out_shape=jax.ShapeDtypeStruct((GRID, 1, 1), jnp.float32),
    )(pred, label.reshape(GRID, 1, BLK), score.reshape(GRID, 1, BLK))
    return (jnp.sum(parts) * (LOSS_WEIGHT / (N * C))).astype(jnp.float32)


# speedup vs baseline: 1.0017x; 1.0017x over previous
import jax
import jax.numpy as jnp
from jax.experimental import pallas as pl

N = 100000
C = 80
LOSS_WEIGHT = 1.0

_LOG1P = (
    0.000925183135894625,
    0.9797525353876084,
    -0.39353455735239506,
    0.10668430401703402,
)

BLK = 4000
GRID = N // BLK
SUB = 80
NSUB = BLK // SUB


def _mxu_col(row):
    return jax.lax.dot_general(
        row, jnp.ones((1, 1), jnp.float32),
        ((( 0,), (0,)), ((), ())),
        preferred_element_type=jnp.float32)


def _body(x_ref, lab_ref, sc_ref, out_ref):
    cols = jax.lax.broadcasted_iota(jnp.int32, (SUB, C), 1).astype(jnp.float32)

    lab_col = _mxu_col(lab_ref[0, :, :].astype(jnp.float32))
    sc_col = _mxu_col(sc_ref[0, :, :])

    acc = jnp.zeros((SUB, C), jnp.float32)
    for j in range(NSUB):
        x = x_ref[pl.ds(j * SUB, SUB), :]
        lab = jax.lax.slice(lab_col, (j * SUB, 0), (j * SUB + SUB, 1))
        sc = jax.lax.slice(sc_col, (j * SUB, 0), (j * SUB + SUB, 1))

        u = jnp.exp(-jnp.abs(x))
        p = jnp.float32(_LOG1P[-1])
        for c in _LOG1P[-2::-1]:
            p = p * u + jnp.float32(c)
        sp = jnp.maximum(x, 0.0) + p
        t = 1.0 / (1.0 + u)
        s = jnp.where(x >= 0, t, u * t)
        negv = (0.75 * sp) * (s * s)

        w = sc - x
        pos_loss = (sp - x * sc) * (w * w)
        onehot = cols == lab
        acc = acc + negv + jnp.where(onehot, pos_loss - negv, 0.0)

    out_ref[...] = jnp.sum(acc).reshape(1, 1, 1)


def kernel(pred, label, score):
    parts = pl.pallas_call(
        _body,
        grid=(GRID,),
        in_specs=[
            pl.BlockSpec((BLK, C), lambda i: (i, 0)),
            pl.BlockSpec((1, 1, BLK), lambda i: (i, 0, 0)),
            pl.BlockSpec((1, 1, BLK), lambda i: (i, 0, 0)),
        ],
        out_specs=pl.BlockSpec((1, 1, 1), lambda i: (i, 0, 0)),
        out_shape=jax.ShapeDtypeStruct((GRID, 1, 1), jnp.float32),
    )(pred, label.reshape(GRID, 1, BLK), score.reshape(GRID, 1, BLK))
    return (jnp.sum(parts) * (LOSS_WEIGHT / (N * C))).astype(jnp.float32)
